# baseline (speedup 1.0000x reference)
import functools

import jax
import jax.numpy as jnp
from jax.experimental import pallas as pl
from jax.experimental.pallas import tpu as pltpu

D_MODEL = 4096
N_EXP = 64
TOP_K = 8
N_TOK = 8192
BLK_T = 512
SUB = 256

_NEG_INF = float("-inf")


def _router_kernel(h_ref, w_ref, mask_ref, probs_ref, logits_ref):
    j = pl.program_id(1)
    hs = h_ref[pl.ds(j * SUB, SUB), :]
    logits = jax.lax.dot_general(
        hs, w_ref[...], (((1,), (1,)), ((), ())),
        preferred_element_type=jnp.float32,
    )
    logits_ref[...] = logits

    idxf = jax.lax.broadcasted_iota(jnp.int32, (N_EXP, SUB), 0).astype(
        jnp.float32
    )
    lg = logits.T
    m = jnp.max(lg, axis=0, keepdims=True)
    e = jnp.exp(lg - m)
    probs_ref[...] = (e / jnp.sum(e, axis=0, keepdims=True)).T
    v = lg
    for _ in range(TOP_K):
        mx = jnp.max(v, axis=0, keepdims=True)
        cand = jnp.where(v == mx, idxf, jnp.float32(N_EXP))
        amin = jnp.min(cand, axis=0, keepdims=True)
        v = jnp.where(idxf == amin, _NEG_INF, v)
    mask_ref[...] = (v == _NEG_INF).T


@functools.partial(jax.jit, static_argnames=())
def kernel(h, W):
    grid = (N_TOK // BLK_T, BLK_T // SUB)
    mask, probs, logits = pl.pallas_call(
        _router_kernel,
        grid=grid,
        in_specs=[
            pl.BlockSpec((BLK_T, D_MODEL), lambda i, j: (i, 0)),
            pl.BlockSpec((N_EXP, D_MODEL), lambda i, j: (0, 0)),
        ],
        out_specs=[
            pl.BlockSpec((SUB, N_EXP), lambda i, j: (i * 2 + j, 0)),
            pl.BlockSpec((SUB, N_EXP), lambda i, j: (i * 2 + j, 0)),
            pl.BlockSpec((SUB, N_EXP), lambda i, j: (i * 2 + j, 0)),
        ],
        out_shape=[
            jax.ShapeDtypeStruct((N_TOK, N_EXP), jnp.bool_),
            jax.ShapeDtypeStruct((N_TOK, N_EXP), jnp.float32),
            jax.ShapeDtypeStruct((N_TOK, N_EXP), jnp.float32),
        ],
        compiler_params=pltpu.CompilerParams(
            dimension_semantics=("parallel", "arbitrary"),
        ),
    )(h, W)
    return (mask, probs, probs, logits)


# speedup vs baseline: 1.0009x; 1.0009x over previous
import functools

import jax
import jax.numpy as jnp
from jax.experimental import pallas as pl
from jax.experimental.pallas import tpu as pltpu

D_MODEL = 4096
N_EXP = 64
TOP_K = 8
N_TOK = 8192
BLK_T = 512
SUB = 256

_NEG_INF = float("-inf")


def _router_kernel(h_ref, w_ref, mask_ref, probs_ref, logits_ref):
    j = pl.program_id(1)
    hs = h_ref[pl.ds(j * SUB, SUB), :]
    logits = jax.lax.dot_general(
        hs, w_ref[...], (((1,), (1,)), ((), ())),
        preferred_element_type=jnp.float32,
    )
    logits_ref[...] = logits

    idxf = jax.lax.broadcasted_iota(jnp.int32, (N_EXP, SUB), 0).astype(
        jnp.float32
    )
    lg = logits.T
    m = jnp.max(lg, axis=0, keepdims=True)
    e = jnp.exp(lg - m)
    probs_ref[...] = (e / jnp.sum(e, axis=0, keepdims=True)).T
    v = lg
    for _ in range(TOP_K):
        mx = jnp.max(v, axis=0, keepdims=True)
        cand = jnp.where(v == mx, idxf, jnp.float32(N_EXP))
        amin = jnp.min(cand, axis=0, keepdims=True)
        v = jnp.where(idxf == amin, _NEG_INF, v)
    mask_ref[...] = (v == _NEG_INF).T


@functools.partial(jax.jit, static_argnames=())
def kernel(h, W):
    grid = (N_TOK // BLK_T, BLK_T // SUB)
    mask, probs, logits = pl.pallas_call(
        _router_kernel,
        grid=grid,
        in_specs=[
            pl.BlockSpec((BLK_T, D_MODEL), lambda i, j: (i, 0)),
            pl.BlockSpec((N_EXP, D_MODEL), lambda i, j: (0, 0)),
        ],
        out_specs=[
            pl.BlockSpec((SUB, N_EXP), lambda i, j: (i * 2 + j, 0)),
            pl.BlockSpec((SUB, N_EXP), lambda i, j: (i * 2 + j, 0)),
            pl.BlockSpec((SUB, N_EXP), lambda i, j: (i * 2 + j, 0)),
        ],
        out_shape=[
            jax.ShapeDtypeStruct((N_TOK, N_EXP), jnp.bool_),
            jax.ShapeDtypeStruct((N_TOK, N_EXP), jnp.float32),
            jax.ShapeDtypeStruct((N_TOK, N_EXP), jnp.float32),
        ],
        compiler_params=pltpu.CompilerParams(
            dimension_semantics=("parallel", "parallel"),
        ),
    )(h, W)
    return (mask, probs, probs, logits)


# speedup vs baseline: 1.5009x; 1.4996x over previous
import functools

import jax
import jax.numpy as jnp
from jax.experimental import pallas as pl
from jax.experimental.pallas import tpu as pltpu

D_MODEL = 4096
N_EXP = 64
TOP_K = 8
N_TOK = 8192
BLK_T = 512
CHUNK = 256

_NEG_INF = float("-inf")


def _router_kernel(h_ref, w_ref, mask_ref, probs_ref, logits_ref):
    logits = jax.lax.dot_general(
        h_ref[...], w_ref[...], (((1,), (1,)), ((), ())),
        preferred_element_type=jnp.float32,
    )
    logits_ref[...] = logits

    idxf = jax.lax.broadcasted_iota(jnp.int32, (N_EXP, CHUNK), 0).astype(
        jnp.float32
    )
    for c in range(BLK_T // CHUNK):
        sl = pl.ds(c * CHUNK, CHUNK)
        lg = logits[c * CHUNK:(c + 1) * CHUNK, :].T
        m = jnp.max(lg, axis=0, keepdims=True)
        e = jnp.exp(lg - m)
        probs_ref[sl, :] = (e / jnp.sum(e, axis=0, keepdims=True)).T
        v = lg
        for _ in range(TOP_K):
            mx = jnp.max(v, axis=0, keepdims=True)
            cand = jnp.where(v == mx, idxf, jnp.float32(N_EXP))
            amin = jnp.min(cand, axis=0, keepdims=True)
            v = jnp.where(idxf == amin, _NEG_INF, v)
        mask_ref[sl, :] = (v == _NEG_INF).T


@functools.partial(jax.jit, static_argnames=())
def kernel(h, W):
    grid = (N_TOK // BLK_T,)
    mask, probs, logits = pl.pallas_call(
        _router_kernel,
        grid=grid,
        in_specs=[
            pl.BlockSpec((BLK_T, D_MODEL), lambda i: (i, 0)),
            pl.BlockSpec((N_EXP, D_MODEL), lambda i: (0, 0)),
        ],
        out_specs=[
            pl.BlockSpec((BLK_T, N_EXP), lambda i: (i, 0)),
            pl.BlockSpec((BLK_T, N_EXP), lambda i: (i, 0)),
            pl.BlockSpec((BLK_T, N_EXP), lambda i: (i, 0)),
        ],
        out_shape=[
            jax.ShapeDtypeStruct((N_TOK, N_EXP), jnp.bool_),
            jax.ShapeDtypeStruct((N_TOK, N_EXP), jnp.float32),
            jax.ShapeDtypeStruct((N_TOK, N_EXP), jnp.float32),
        ],
        compiler_params=pltpu.CompilerParams(
            dimension_semantics=("parallel",),
        ),
    )(h, W)
    return (mask, probs, probs, logits)
